# baseline (speedup 1.0000x reference)
import functools

import jax
import jax.numpy as jnp
import numpy as np
from jax import lax
from jax.experimental import pallas as pl
from jax.experimental.pallas import tpu as pltpu
from jax.experimental.pallas import tpu_sc as plsc

_N = 10000
_E = 320000
_F = 128
_EDGE_DIM = 16
_NUM_GRAPHS = 64
_AVG_LOG_DEG = float(np.log(33.0))
_HI = jax.lax.Precision.HIGHEST


def _dotd(a, b):
    return jax.lax.dot_general(
        a.astype(jnp.bfloat16), b.astype(jnp.bfloat16),
        (((1,), (0,)), ((), ())), preferred_element_type=jnp.float32)

_ROW_BLK = 1000
_EDGE_BLK = 4000


def _a1_body(x_ref, w1_ref, b1_ref, w2_ref, b2_ref, wpd_ref, wps_ref,
             h_ref, hd_ref, hs_ref):
    x = x_ref[...]
    hmid = jnp.maximum(_dotd(x, w1_ref[...]) + b1_ref[...], 0.0)
    h = _dotd(hmid, w2_ref[...]) + b2_ref[...]
    h_ref[...] = h
    hd_ref[...] = _dotd(h, wpd_ref[...])
    hs_ref[...] = _dotd(h, wps_ref[...])


def _run_a1(x, W1, b1, W2, b2, Wpd, Wps):
    nblk = _N // _ROW_BLK
    row_spec = pl.BlockSpec((_ROW_BLK, _F), lambda i: (i, 0))
    full = lambda a: pl.BlockSpec(a.shape, lambda i: (0,) * a.ndim)
    out_sd = jax.ShapeDtypeStruct((_N, _F), jnp.float32)
    return pl.pallas_call(
        _a1_body,
        grid=(nblk,),
        in_specs=[row_spec, full(W1), full(b1), full(W2), full(b2),
                  full(Wpd), full(Wps)],
        out_specs=[row_spec, row_spec, row_spec],
        out_shape=[out_sd, out_sd, out_sd],
    )(x, W1, b1, W2, b2, Wpd, Wps)


def _a2_body(ea_ref, we_ref, wpe_ref, be_ref, bp_ref, et_ref):
    e = _dotd(ea_ref[...], we_ref[...]) + be_ref[...]
    et_ref[...] = _dotd(e, wpe_ref[...]) + bp_ref[...]


def _run_a2(edge_attr, We, Wpe, be, bp):
    nblk = _E // _EDGE_BLK
    full = lambda a: pl.BlockSpec(a.shape, lambda i: (0,) * a.ndim)
    return pl.pallas_call(
        _a2_body,
        grid=(nblk,),
        in_specs=[pl.BlockSpec((_EDGE_BLK, _EDGE_DIM), lambda i: (i, 0)),
                  full(We), full(Wpe), full(be), full(bp)],
        out_specs=pl.BlockSpec((_EDGE_BLK, _F), lambda i: (i, 0)),
        out_shape=jax.ShapeDtypeStruct((_E, _F), jnp.float32),
    )(edge_attr, We, Wpe, be, bp)


def _c_body(h_ref, hd_ref, cnt_ref, sum_ref, sq_ref, mx_ref, mn_ref,
            batch_ref, wpost_ref, bpost_ref, g1_ref, beta1_ref,
            wa_ref, ba_ref, wb_ref, bb_ref, out_ref, pooled_ref):
    i = pl.program_id(0)
    nblk = pl.num_programs(0)

    h = h_ref[...]
    hd = hd_ref[...]
    cnt = cnt_ref[...]
    sum_q = sum_ref[...]
    sq_q = sq_ref[...]
    cnt_c = jnp.maximum(cnt, 1.0)
    sum_m = sum_q + cnt * hd
    mean = sum_m / cnt_c
    mean_sq = (sq_q + 2.0 * hd * sum_q + cnt * hd * hd) / cnt_c
    std = jnp.sqrt(jnp.maximum(mean_sq - mean * mean, 0.0) + 1e-5)
    has = cnt > 0.0
    mx = jnp.where(has, hd + mx_ref[...], 0.0)
    mn = jnp.where(has, hd + mn_ref[...], 0.0)
    aggs = jnp.concatenate([mean, mn, mx, std], axis=-1)
    logd = jnp.log(cnt + 1.0)
    amp = logd / _AVG_LOG_DEG
    safe_logd = jnp.where(logd > 0.0, logd, 1.0)
    att = jnp.where(logd > 0.0, _AVG_LOG_DEG / safe_logd, 1.0)
    scaled = jnp.concatenate([aggs, aggs * amp, aggs * att], axis=-1)
    h2 = jnp.concatenate([h, scaled], axis=-1)
    h2 = _dotd(h2, wpost_ref[...]) + bpost_ref[...]
    h2 = g1_ref[...] * h2 / np.sqrt(1.0 + 1e-5) + beta1_ref[...]
    h2 = jnp.maximum(h2, 0.0)

    batch = batch_ref[0, 0, :]
    gids = jax.lax.broadcasted_iota(jnp.int32, (_NUM_GRAPHS, h.shape[0]), 0)
    onehot = (gids == batch[None, :]).astype(jnp.float32)
    part = jax.lax.dot(onehot, h2, precision=_HI)

    @pl.when(i == 0)
    def _():
        pooled_ref[...] = jnp.zeros_like(pooled_ref)

    pooled_ref[...] += part

    @pl.when(i == nblk - 1)
    def _():
        pooled = pooled_ref[...]
        a = jnp.maximum(_dotd(pooled, wa_ref[...]) + ba_ref[...], 0.0)
        out_ref[...] = _dotd(a, wb_ref[...]) + bb_ref[...]


def _run_c(h, hd, cnt2d, sum_q, sq_q, mx_q, mn_q, batch3d,
           Wpost, bpost, g1, beta1, Wa_p, ba_p, Wb_p, bb):
    nblk = _N // _ROW_BLK
    row_spec = pl.BlockSpec((_ROW_BLK, _F), lambda i: (i, 0))
    cnt_spec = pl.BlockSpec((_ROW_BLK, 1), lambda i: (i, 0))
    b_spec = pl.BlockSpec((1, 1, _ROW_BLK), lambda i: (i, 0, 0))
    full = lambda a: pl.BlockSpec(a.shape, lambda i: (0,) * a.ndim)
    return pl.pallas_call(
        _c_body,
        grid=(nblk,),
        in_specs=[row_spec, row_spec, cnt_spec, row_spec, row_spec,
                  row_spec, row_spec, b_spec, full(Wpost), full(bpost),
                  full(g1), full(beta1), full(Wa_p), full(ba_p),
                  full(Wb_p), full(bb)],
        out_specs=pl.BlockSpec((_NUM_GRAPHS, 1), lambda i: (0, 0)),
        out_shape=jax.ShapeDtypeStruct((_NUM_GRAPHS, 1), jnp.float32),
        scratch_shapes=[pltpu.VMEM((_NUM_GRAPHS, _F), jnp.float32)],
    )(h, hd, cnt2d, sum_q, sq_q, mx_q, mn_q, batch3d,
      Wpost, bpost, g1, beta1, Wa_p, ba_p, Wb_p, bb)


_NC = 2
_NS = 16
_NW = _NC * _NS
_RS = 160
_NPAD = _NW * 2 * _RS
_CHUNK = 2000
_NCHUNK = _E // _CHUNK
_LB = 4096
_FLUSH = 2048
_RING = 6


def _b_body(dst_hbm, src_hbm, hs_hbm, et_hbm,
            cnt_hbm, sum_hbm, sq_hbm, mx_hbm, mn_hbm,
            dst_c, src_c, dloc_b, srcm_b, eid_b, hs_rows, et_rows,
            acc_sum, acc_sq, acc_mx, acc_mn, cnt_acc, sem_a, sem_b):
    wid = lax.axis_index("s") * _NC + lax.axis_index("c")
    zeros16 = jnp.zeros((16,), jnp.float32)
    ones16 = jnp.ones((16,), jnp.float32)
    neg16 = jnp.full((16,), -jnp.inf, jnp.float32)
    pos16 = jnp.full((16,), jnp.inf, jnp.float32)
    zi16 = jnp.zeros((16,), jnp.int32)
    iota16 = lax.iota(jnp.int32, 16)

    def fire(b):
        s = lax.rem(b, _RING)
        iva = srcm_b[pl.ds(b * 16, 16)]
        ivb = eid_b[pl.ds(b * 16, 16)]
        pltpu.async_copy(hs_hbm.at[iva], hs_rows.at[s], sem_a)
        pltpu.async_copy(et_hbm.at[ivb], et_rows.at[s], sem_b)

    def flush(off):
        nb = (off + 15) // 16
        srcm_b[pl.ds(off, 16)] = zi16
        eid_b[pl.ds(off, 16)] = zi16

        def prefire(b, _):
            fire(b)
            return 0

        lax.fori_loop(0, jnp.minimum(nb, _RING), prefire, 0)

        def batch_body(b, _):
            s = lax.rem(b, _RING)
            bb0 = b * 16
            ivd = srcm_b[pl.ds(bb0, 16)]
            pltpu.make_async_copy(hs_hbm.at[ivd], hs_rows.at[s], sem_a).wait()
            pltpu.make_async_copy(et_hbm.at[ivd], et_rows.at[s], sem_b).wait()
            ne = jnp.minimum(16, off - bb0)

            def edge_body(j, _):
                dloc = dloc_b[pl.ds(bb0 + j, 16)][0]
                rowb = dloc * _F
                for vi in range(_F // 16):
                    fs = pl.ds(vi * 16, 16)
                    asl = pl.ds(rowb + vi * 16, 16)
                    q = hs_rows[s, j, fs] + et_rows[s, j, fs]
                    acc_sum[asl] += q
                    acc_sq[asl] += q * q
                    acc_mx[asl] = jnp.maximum(acc_mx[asl], q)
                    acc_mn[asl] = jnp.minimum(acc_mn[asl], q)
                return 0

            lax.fori_loop(0, ne, edge_body, 0)

            @pl.when(b + _RING < nb)
            def _():
                fire(b + _RING)

            return 0

        lax.fori_loop(0, nb, batch_body, 0)

    def flush_reset(off):
        flush(off)
        return jnp.int32(0)

    for r_i in range(2):
        r = wid * 2 + r_i
        lo = r * _RS
        hi = lo + _RS

        def init_body(k, _):
            sl = pl.ds(k * 16, 16)
            acc_sum[sl] = zeros16
            acc_sq[sl] = zeros16
            acc_mx[sl] = neg16
            acc_mn[sl] = pos16
            return 0

        lax.fori_loop(0, _RS * _F // 16, init_body, 0)

        def cinit_body(k, _):
            cnt_acc[pl.ds(k * 16, 16)] = zeros16
            return 0

        lax.fori_loop(0, _RS // 16, cinit_body, 0)

        def chunk_body(c, off):
            base = c * _CHUNK
            cpd = pltpu.async_copy(dst_hbm.at[pl.ds(base, _CHUNK)], dst_c, sem_a)
            cps = pltpu.async_copy(src_hbm.at[pl.ds(base, _CHUNK)], src_c, sem_b)
            cpd.wait()
            cps.wait()

            def scan_body(v, off):
                sl = pl.ds(v * 16, 16)
                d = dst_c[sl]
                msk = (d >= lo) & (d < hi)

                def compact(off):
                    s = src_c[sl]
                    dl = d - lo
                    cs = plsc.cumsum(msk.astype(jnp.int32))
                    pos = off + cs - 1
                    plsc.store_scatter(dloc_b, [pos], dl, mask=msk)
                    plsc.store_scatter(srcm_b, [pos], s, mask=msk)
                    eid = (base + v * 16) + iota16
                    plsc.store_scatter(eid_b, [pos], eid, mask=msk)
                    plsc.addupdate_scatter(cnt_acc, [dl], ones16, mask=msk)
                    return off + cs[15]

                return lax.cond(jnp.any(msk), compact, lambda o: o, off)

            off = lax.fori_loop(0, _CHUNK // 16, scan_body, off)
            return lax.cond(off >= _FLUSH, flush_reset, lambda o: o, off)

        off_fin = lax.fori_loop(0, _NCHUNK, chunk_body, jnp.int32(0))
        lax.cond(off_fin > 0, flush_reset, lambda o: o, off_fin)

        pltpu.sync_copy(cnt_acc, cnt_hbm.at[pl.ds(lo, _RS)])
        pltpu.sync_copy(acc_sum, sum_hbm.at[pl.ds(lo * _F, _RS * _F)])
        pltpu.sync_copy(acc_sq, sq_hbm.at[pl.ds(lo * _F, _RS * _F)])
        pltpu.sync_copy(acc_mx, mx_hbm.at[pl.ds(lo * _F, _RS * _F)])
        pltpu.sync_copy(acc_mn, mn_hbm.at[pl.ds(lo * _F, _RS * _F)])


def _run_b(dst, src, hs, et):
    kern = pl.kernel(
        _b_body,
        out_type=[
            jax.ShapeDtypeStruct((_NPAD,), jnp.float32),
            jax.ShapeDtypeStruct((_NPAD * _F,), jnp.float32),
            jax.ShapeDtypeStruct((_NPAD * _F,), jnp.float32),
            jax.ShapeDtypeStruct((_NPAD * _F,), jnp.float32),
            jax.ShapeDtypeStruct((_NPAD * _F,), jnp.float32),
        ],
        mesh=plsc.VectorSubcoreMesh(
            core_axis_name="c", subcore_axis_name="s",
            num_cores=_NC, num_subcores=_NS),
        compiler_params=pltpu.CompilerParams(needs_layout_passes=False),
        scratch_types=[
            pltpu.VMEM((_CHUNK,), jnp.int32),
            pltpu.VMEM((_CHUNK,), jnp.int32),
            pltpu.VMEM((_LB,), jnp.int32),
            pltpu.VMEM((_LB,), jnp.int32),
            pltpu.VMEM((_LB,), jnp.int32),
            pltpu.VMEM((_RING, 16, _F), jnp.float32),
            pltpu.VMEM((_RING, 16, _F), jnp.float32),
            pltpu.VMEM((_RS * _F,), jnp.float32),
            pltpu.VMEM((_RS * _F,), jnp.float32),
            pltpu.VMEM((_RS * _F,), jnp.float32),
            pltpu.VMEM((_RS * _F,), jnp.float32),
            pltpu.VMEM((_RS,), jnp.float32),
            pltpu.SemaphoreType.DMA,
            pltpu.SemaphoreType.DMA,
        ],
    )
    cnt_p, sum_p, sq_p, mx_p, mn_p = kern(dst, src, hs, et)
    cnt = cnt_p[:_N]
    sum_q = sum_p.reshape(_NPAD, _F)[:_N]
    sq_q = sq_p.reshape(_NPAD, _F)[:_N]
    mx_q = mx_p.reshape(_NPAD, _F)[:_N]
    mn_q = mn_p.reshape(_NPAD, _F)[:_N]
    return cnt, sum_q, sq_q, mx_q, mn_q


def kernel(x, edge_index, edge_attr, batch, W1, b1, W2, b2, We, be, Wp, bp,
           Wpost, bpost, g1, beta1, Wa, ba, Wb, bb):
    Wpd = Wp[:_F]
    Wps = Wp[_F:2 * _F]
    Wpe = Wp[2 * _F:]
    b1r = b1.reshape(1, -1)
    b2r = b2.reshape(1, -1)
    ber = be.reshape(1, -1)
    bpr = bp.reshape(1, -1)
    bpostr = bpost.reshape(1, -1)
    g1r = g1.reshape(1, -1)
    beta1r = beta1.reshape(1, -1)
    Wa_p = jnp.pad(Wa, ((0, 0), (0, _F - Wa.shape[1])))
    ba_p = jnp.pad(ba, ((0, _F - ba.shape[0]))).reshape(1, -1)
    Wb_p = jnp.pad(Wb, ((0, _F - Wb.shape[0]), (0, 0)))
    bbr = bb.reshape(1, -1)
    batch3d = batch.reshape(_N // _ROW_BLK, 1, _ROW_BLK)

    h, hd, hs = _run_a1(x, W1, b1r, W2, b2r, Wpd, Wps)
    et = _run_a2(edge_attr, We, Wpe, ber, bpr)
    cnt, sum_q, sq_q, mx_q, mn_q = _run_b(edge_index[1], edge_index[0], hs, et)
    out = _run_c(h, hd, cnt.reshape(_N, 1), sum_q, sq_q, mx_q, mn_q,
                 batch3d, Wpost, bpostr, g1r, beta1r, Wa_p, ba_p, Wb_p, bbr)
    return out


# speedup vs baseline: 1.0353x; 1.0353x over previous
import functools

import jax
import jax.numpy as jnp
import numpy as np
from jax import lax
from jax.experimental import pallas as pl
from jax.experimental.pallas import tpu as pltpu
from jax.experimental.pallas import tpu_sc as plsc

_N = 10000
_E = 320000
_F = 128
_EDGE_DIM = 16
_NUM_GRAPHS = 64
_AVG_LOG_DEG = float(np.log(33.0))
_HI = jax.lax.Precision.HIGHEST


def _dotd(a, b):
    return jax.lax.dot_general(
        a.astype(jnp.bfloat16), b.astype(jnp.bfloat16),
        (((1,), (0,)), ((), ())), preferred_element_type=jnp.float32)

_ROW_BLK = 1000
_EDGE_BLK = 4000


def _a1_body(x_ref, w1_ref, b1_ref, w2_ref, b2_ref, wpd_ref, wps_ref,
             h_ref, hd_ref, hs_ref):
    x = x_ref[...]
    hmid = jnp.maximum(_dotd(x, w1_ref[...]) + b1_ref[...], 0.0)
    h = _dotd(hmid, w2_ref[...]) + b2_ref[...]
    h_ref[...] = h
    hd_ref[...] = _dotd(h, wpd_ref[...])
    hs_ref[...] = _dotd(h, wps_ref[...])


def _run_a1(x, W1, b1, W2, b2, Wpd, Wps):
    nblk = _N // _ROW_BLK
    row_spec = pl.BlockSpec((_ROW_BLK, _F), lambda i: (i, 0))
    full = lambda a: pl.BlockSpec(a.shape, lambda i: (0,) * a.ndim)
    out_sd = jax.ShapeDtypeStruct((_N, _F), jnp.float32)
    return pl.pallas_call(
        _a1_body,
        grid=(nblk,),
        in_specs=[row_spec, full(W1), full(b1), full(W2), full(b2),
                  full(Wpd), full(Wps)],
        out_specs=[row_spec, row_spec, row_spec],
        out_shape=[out_sd, out_sd, out_sd],
    )(x, W1, b1, W2, b2, Wpd, Wps)


def _a2_body(ea_ref, we_ref, wpe_ref, be_ref, bp_ref, et_ref):
    e = _dotd(ea_ref[...], we_ref[...]) + be_ref[...]
    et_ref[...] = _dotd(e, wpe_ref[...]) + bp_ref[...]


def _run_a2(edge_attr, We, Wpe, be, bp):
    nblk = _E // _EDGE_BLK
    full = lambda a: pl.BlockSpec(a.shape, lambda i: (0,) * a.ndim)
    return pl.pallas_call(
        _a2_body,
        grid=(nblk,),
        in_specs=[pl.BlockSpec((_EDGE_BLK, _EDGE_DIM), lambda i: (i, 0)),
                  full(We), full(Wpe), full(be), full(bp)],
        out_specs=pl.BlockSpec((_EDGE_BLK, _F), lambda i: (i, 0)),
        out_shape=jax.ShapeDtypeStruct((_E, _F), jnp.float32),
    )(edge_attr, We, Wpe, be, bp)


def _c_body(h_ref, hd_ref, cnt_ref, sum_ref, sq_ref, mx_ref, mn_ref,
            batch_ref, wpost_ref, bpost_ref, g1_ref, beta1_ref,
            wa_ref, ba_ref, wb_ref, bb_ref, out_ref, pooled_ref):
    i = pl.program_id(0)
    nblk = pl.num_programs(0)

    h = h_ref[...]
    hd = hd_ref[...]
    cnt = cnt_ref[...]
    sum_q = sum_ref[...]
    sq_q = sq_ref[...]
    cnt_c = jnp.maximum(cnt, 1.0)
    sum_m = sum_q + cnt * hd
    mean = sum_m / cnt_c
    mean_sq = (sq_q + 2.0 * hd * sum_q + cnt * hd * hd) / cnt_c
    std = jnp.sqrt(jnp.maximum(mean_sq - mean * mean, 0.0) + 1e-5)
    has = cnt > 0.0
    mx = jnp.where(has, hd + mx_ref[...], 0.0)
    mn = jnp.where(has, hd + mn_ref[...], 0.0)
    aggs = jnp.concatenate([mean, mn, mx, std], axis=-1)
    logd = jnp.log(cnt + 1.0)
    amp = logd / _AVG_LOG_DEG
    safe_logd = jnp.where(logd > 0.0, logd, 1.0)
    att = jnp.where(logd > 0.0, _AVG_LOG_DEG / safe_logd, 1.0)
    scaled = jnp.concatenate([aggs, aggs * amp, aggs * att], axis=-1)
    h2 = jnp.concatenate([h, scaled], axis=-1)
    h2 = _dotd(h2, wpost_ref[...]) + bpost_ref[...]
    h2 = g1_ref[...] * h2 / np.sqrt(1.0 + 1e-5) + beta1_ref[...]
    h2 = jnp.maximum(h2, 0.0)

    batch = batch_ref[0, 0, :]
    gids = jax.lax.broadcasted_iota(jnp.int32, (_NUM_GRAPHS, h.shape[0]), 0)
    onehot = (gids == batch[None, :]).astype(jnp.float32)
    part = jax.lax.dot(onehot, h2, precision=_HI)

    @pl.when(i == 0)
    def _():
        pooled_ref[...] = jnp.zeros_like(pooled_ref)

    pooled_ref[...] += part

    @pl.when(i == nblk - 1)
    def _():
        pooled = pooled_ref[...]
        a = jnp.maximum(_dotd(pooled, wa_ref[...]) + ba_ref[...], 0.0)
        out_ref[...] = _dotd(a, wb_ref[...]) + bb_ref[...]


def _run_c(h, hd, cnt2d, sum_q, sq_q, mx_q, mn_q, batch3d,
           Wpost, bpost, g1, beta1, Wa_p, ba_p, Wb_p, bb):
    nblk = _N // _ROW_BLK
    row_spec = pl.BlockSpec((_ROW_BLK, _F), lambda i: (i, 0))
    cnt_spec = pl.BlockSpec((_ROW_BLK, 1), lambda i: (i, 0))
    b_spec = pl.BlockSpec((1, 1, _ROW_BLK), lambda i: (i, 0, 0))
    full = lambda a: pl.BlockSpec(a.shape, lambda i: (0,) * a.ndim)
    return pl.pallas_call(
        _c_body,
        grid=(nblk,),
        in_specs=[row_spec, row_spec, cnt_spec, row_spec, row_spec,
                  row_spec, row_spec, b_spec, full(Wpost), full(bpost),
                  full(g1), full(beta1), full(Wa_p), full(ba_p),
                  full(Wb_p), full(bb)],
        out_specs=pl.BlockSpec((_NUM_GRAPHS, 1), lambda i: (0, 0)),
        out_shape=jax.ShapeDtypeStruct((_NUM_GRAPHS, 1), jnp.float32),
        scratch_shapes=[pltpu.VMEM((_NUM_GRAPHS, _F), jnp.float32)],
    )(h, hd, cnt2d, sum_q, sq_q, mx_q, mn_q, batch3d,
      Wpost, bpost, g1, beta1, Wa_p, ba_p, Wb_p, bb)


_NC = 2
_NS = 16
_NW = _NC * _NS
_RS = 160
_NPAD = _NW * 2 * _RS
_CHUNK = 2000
_NCHUNK = _E // _CHUNK
_LB = 4096
_FLUSH = 2048
_RING = 4


def _b_body(dst_hbm, src_hbm, hs_hbm, et_hbm,
            cnt_hbm, sum_hbm, sq_hbm, mx_hbm, mn_hbm,
            dst_c, src_c, dloc_b, srcm_b, eid_b, hs_rows, et_rows,
            acc_sum, acc_sq, acc_mx, acc_mn, cnt_acc, sem_a, sem_b):
    wid = lax.axis_index("s") * _NC + lax.axis_index("c")
    zeros16 = jnp.zeros((16,), jnp.float32)
    ones16 = jnp.ones((16,), jnp.float32)
    neg16 = jnp.full((16,), -jnp.inf, jnp.float32)
    pos16 = jnp.full((16,), jnp.inf, jnp.float32)
    zi16 = jnp.zeros((16,), jnp.int32)
    iota16 = lax.iota(jnp.int32, 16)

    def fire(b):
        s = lax.rem(b, _RING)
        iva = srcm_b[pl.ds(b * 16, 16)]
        ivb = eid_b[pl.ds(b * 16, 16)]
        pltpu.async_copy(hs_hbm.at[iva], hs_rows.at[s], sem_a)
        pltpu.async_copy(et_hbm.at[ivb], et_rows.at[s], sem_b)

    def flush(off):
        nb = (off + 15) // 16
        srcm_b[pl.ds(off, 16)] = zi16
        eid_b[pl.ds(off, 16)] = zi16

        def prefire(b, _):
            fire(b)
            return 0

        lax.fori_loop(0, jnp.minimum(nb, _RING), prefire, 0)

        def batch_body(b, _):
            s = lax.rem(b, _RING)
            bb0 = b * 16
            ivd = srcm_b[pl.ds(bb0, 16)]
            pltpu.make_async_copy(hs_hbm.at[ivd], hs_rows.at[s], sem_a).wait()
            pltpu.make_async_copy(et_hbm.at[ivd], et_rows.at[s], sem_b).wait()
            ne = jnp.minimum(16, off - bb0)

            def edge_body(j, _):
                dloc = dloc_b[pl.ds(bb0 + j, 16)][0]
                rowb = dloc * _F
                for vi in range(_F // 16):
                    fs = pl.ds(vi * 16, 16)
                    asl = pl.ds(rowb + vi * 16, 16)
                    q = hs_rows[s, j, fs] + et_rows[s, j, fs]
                    plsc.addupdate(acc_sum.at[asl], q)
                    plsc.addupdate(acc_sq.at[asl], q * q)
                    acc_mx[asl] = jnp.maximum(acc_mx[asl], q)
                    acc_mn[asl] = jnp.minimum(acc_mn[asl], q)
                return 0

            lax.fori_loop(0, ne, edge_body, 0)

            @pl.when(b + _RING < nb)
            def _():
                fire(b + _RING)

            return 0

        lax.fori_loop(0, nb, batch_body, 0)

    def flush_reset(off):
        flush(off)
        return jnp.int32(0)

    for r_i in range(2):
        r = wid * 2 + r_i
        lo = r * _RS
        hi = lo + _RS

        def init_body(k, _):
            sl = pl.ds(k * 16, 16)
            acc_sum[sl] = zeros16
            acc_sq[sl] = zeros16
            acc_mx[sl] = neg16
            acc_mn[sl] = pos16
            return 0

        lax.fori_loop(0, _RS * _F // 16, init_body, 0)

        def cinit_body(k, _):
            cnt_acc[pl.ds(k * 16, 16)] = zeros16
            return 0

        lax.fori_loop(0, _RS // 16, cinit_body, 0)

        def chunk_body(c, off):
            base = c * _CHUNK
            cpd = pltpu.async_copy(dst_hbm.at[pl.ds(base, _CHUNK)], dst_c, sem_a)
            cps = pltpu.async_copy(src_hbm.at[pl.ds(base, _CHUNK)], src_c, sem_b)
            cpd.wait()
            cps.wait()

            def scan_body(v, off):
                sl = pl.ds(v * 16, 16)
                d = dst_c[sl]
                msk = (d >= lo) & (d < hi)

                def compact(off):
                    s = src_c[sl]
                    dl = d - lo
                    cs = plsc.cumsum(msk.astype(jnp.int32))
                    pos = off + cs - 1
                    plsc.store_scatter(dloc_b, [pos], dl, mask=msk)
                    plsc.store_scatter(srcm_b, [pos], s, mask=msk)
                    eid = (base + v * 16) + iota16
                    plsc.store_scatter(eid_b, [pos], eid, mask=msk)
                    plsc.addupdate_scatter(cnt_acc, [dl], ones16, mask=msk)
                    return off + cs[15]

                return lax.cond(jnp.any(msk), compact, lambda o: o, off)

            off = lax.fori_loop(0, _CHUNK // 16, scan_body, off)
            return lax.cond(off >= _FLUSH, flush_reset, lambda o: o, off)

        off_fin = lax.fori_loop(0, _NCHUNK, chunk_body, jnp.int32(0))
        lax.cond(off_fin > 0, flush_reset, lambda o: o, off_fin)

        pltpu.sync_copy(cnt_acc, cnt_hbm.at[pl.ds(lo, _RS)])
        pltpu.sync_copy(acc_sum, sum_hbm.at[pl.ds(lo * _F, _RS * _F)])
        pltpu.sync_copy(acc_sq, sq_hbm.at[pl.ds(lo * _F, _RS * _F)])
        pltpu.sync_copy(acc_mx, mx_hbm.at[pl.ds(lo * _F, _RS * _F)])
        pltpu.sync_copy(acc_mn, mn_hbm.at[pl.ds(lo * _F, _RS * _F)])


def _run_b(dst, src, hs, et):
    kern = pl.kernel(
        _b_body,
        out_type=[
            jax.ShapeDtypeStruct((_NPAD,), jnp.float32),
            jax.ShapeDtypeStruct((_NPAD * _F,), jnp.float32),
            jax.ShapeDtypeStruct((_NPAD * _F,), jnp.float32),
            jax.ShapeDtypeStruct((_NPAD * _F,), jnp.float32),
            jax.ShapeDtypeStruct((_NPAD * _F,), jnp.float32),
        ],
        mesh=plsc.VectorSubcoreMesh(
            core_axis_name="c", subcore_axis_name="s",
            num_cores=_NC, num_subcores=_NS),
        compiler_params=pltpu.CompilerParams(needs_layout_passes=False),
        scratch_types=[
            pltpu.VMEM((_CHUNK,), jnp.int32),
            pltpu.VMEM((_CHUNK,), jnp.int32),
            pltpu.VMEM((_LB,), jnp.int32),
            pltpu.VMEM((_LB,), jnp.int32),
            pltpu.VMEM((_LB,), jnp.int32),
            pltpu.VMEM((_RING, 16, _F), jnp.float32),
            pltpu.VMEM((_RING, 16, _F), jnp.float32),
            pltpu.VMEM((_RS * _F,), jnp.float32),
            pltpu.VMEM((_RS * _F,), jnp.float32),
            pltpu.VMEM((_RS * _F,), jnp.float32),
            pltpu.VMEM((_RS * _F,), jnp.float32),
            pltpu.VMEM((_RS,), jnp.float32),
            pltpu.SemaphoreType.DMA,
            pltpu.SemaphoreType.DMA,
        ],
    )
    cnt_p, sum_p, sq_p, mx_p, mn_p = kern(dst, src, hs, et)
    cnt = cnt_p[:_N]
    sum_q = sum_p.reshape(_NPAD, _F)[:_N]
    sq_q = sq_p.reshape(_NPAD, _F)[:_N]
    mx_q = mx_p.reshape(_NPAD, _F)[:_N]
    mn_q = mn_p.reshape(_NPAD, _F)[:_N]
    return cnt, sum_q, sq_q, mx_q, mn_q


def kernel(x, edge_index, edge_attr, batch, W1, b1, W2, b2, We, be, Wp, bp,
           Wpost, bpost, g1, beta1, Wa, ba, Wb, bb):
    Wpd = Wp[:_F]
    Wps = Wp[_F:2 * _F]
    Wpe = Wp[2 * _F:]
    b1r = b1.reshape(1, -1)
    b2r = b2.reshape(1, -1)
    ber = be.reshape(1, -1)
    bpr = bp.reshape(1, -1)
    bpostr = bpost.reshape(1, -1)
    g1r = g1.reshape(1, -1)
    beta1r = beta1.reshape(1, -1)
    Wa_p = jnp.pad(Wa, ((0, 0), (0, _F - Wa.shape[1])))
    ba_p = jnp.pad(ba, ((0, _F - ba.shape[0]))).reshape(1, -1)
    Wb_p = jnp.pad(Wb, ((0, _F - Wb.shape[0]), (0, 0)))
    bbr = bb.reshape(1, -1)
    batch3d = batch.reshape(_N // _ROW_BLK, 1, _ROW_BLK)

    h, hd, hs = _run_a1(x, W1, b1r, W2, b2r, Wpd, Wps)
    et = _run_a2(edge_attr, We, Wpe, ber, bpr)
    cnt, sum_q, sq_q, mx_q, mn_q = _run_b(edge_index[1], edge_index[0], hs, et)
    out = _run_c(h, hd, cnt.reshape(_N, 1), sum_q, sq_q, mx_q, mn_q,
                 batch3d, Wpost, bpostr, g1r, beta1r, Wa_p, ba_p, Wb_p, bbr)
    return out


# speedup vs baseline: 1.1377x; 1.0989x over previous
import functools

import jax
import jax.numpy as jnp
import numpy as np
from jax import lax
from jax.experimental import pallas as pl
from jax.experimental.pallas import tpu as pltpu
from jax.experimental.pallas import tpu_sc as plsc

_N = 10000
_E = 320000
_F = 128
_EDGE_DIM = 16
_NUM_GRAPHS = 64
_AVG_LOG_DEG = float(np.log(33.0))
_HI = jax.lax.Precision.HIGHEST


def _dotd(a, b):
    return jax.lax.dot_general(
        a.astype(jnp.bfloat16), b.astype(jnp.bfloat16),
        (((1,), (0,)), ((), ())), preferred_element_type=jnp.float32)

_ROW_BLK = 1000
_EDGE_BLK = 4000


def _a1_body(x_ref, w1_ref, b1_ref, w2_ref, b2_ref, wpd_ref, wps_ref,
             h_ref, hd_ref, hs_ref):
    x = x_ref[...]
    hmid = jnp.maximum(_dotd(x, w1_ref[...]) + b1_ref[...], 0.0)
    h = _dotd(hmid, w2_ref[...]) + b2_ref[...]
    h_ref[...] = h
    hd_ref[...] = _dotd(h, wpd_ref[...])
    hs_ref[...] = _dotd(h, wps_ref[...])


def _run_a1(x, W1, b1, W2, b2, Wpd, Wps):
    nblk = _N // _ROW_BLK
    row_spec = pl.BlockSpec((_ROW_BLK, _F), lambda i: (i, 0))
    full = lambda a: pl.BlockSpec(a.shape, lambda i: (0,) * a.ndim)
    out_sd = jax.ShapeDtypeStruct((_N, _F), jnp.float32)
    return pl.pallas_call(
        _a1_body,
        grid=(nblk,),
        in_specs=[row_spec, full(W1), full(b1), full(W2), full(b2),
                  full(Wpd), full(Wps)],
        out_specs=[row_spec, row_spec, row_spec],
        out_shape=[out_sd, out_sd, out_sd],
    )(x, W1, b1, W2, b2, Wpd, Wps)


def _a2_body(ea_ref, we_ref, wpe_ref, be_ref, bp_ref, et_ref):
    e = _dotd(ea_ref[...], we_ref[...]) + be_ref[...]
    et_ref[...] = _dotd(e, wpe_ref[...]) + bp_ref[...]


def _run_a2(edge_attr, We, Wpe, be, bp):
    nblk = _E // _EDGE_BLK
    full = lambda a: pl.BlockSpec(a.shape, lambda i: (0,) * a.ndim)
    return pl.pallas_call(
        _a2_body,
        grid=(nblk,),
        in_specs=[pl.BlockSpec((_EDGE_BLK, _EDGE_DIM), lambda i: (i, 0)),
                  full(We), full(Wpe), full(be), full(bp)],
        out_specs=pl.BlockSpec((_EDGE_BLK, _F), lambda i: (i, 0)),
        out_shape=jax.ShapeDtypeStruct((_E, _F), jnp.float32),
    )(edge_attr, We, Wpe, be, bp)


def _c_body(h_ref, hd_ref, cnt_ref, sum_ref, sq_ref, mx_ref, mn_ref,
            batch_ref, wpost_ref, bpost_ref, g1_ref, beta1_ref,
            wa_ref, ba_ref, wb_ref, bb_ref, out_ref, pooled_ref):
    i = pl.program_id(0)
    nblk = pl.num_programs(0)

    h = h_ref[...]
    hd = hd_ref[...]
    cnt = cnt_ref[...]
    sum_q = sum_ref[...]
    sq_q = sq_ref[...]
    cnt_c = jnp.maximum(cnt, 1.0)
    sum_m = sum_q + cnt * hd
    mean = sum_m / cnt_c
    mean_sq = (sq_q + 2.0 * hd * sum_q + cnt * hd * hd) / cnt_c
    std = jnp.sqrt(jnp.maximum(mean_sq - mean * mean, 0.0) + 1e-5)
    has = cnt > 0.0
    mx = jnp.where(has, hd + mx_ref[...], 0.0)
    mn = jnp.where(has, hd + mn_ref[...], 0.0)
    aggs = jnp.concatenate([mean, mn, mx, std], axis=-1)
    logd = jnp.log(cnt + 1.0)
    amp = logd / _AVG_LOG_DEG
    safe_logd = jnp.where(logd > 0.0, logd, 1.0)
    att = jnp.where(logd > 0.0, _AVG_LOG_DEG / safe_logd, 1.0)
    scaled = jnp.concatenate([aggs, aggs * amp, aggs * att], axis=-1)
    h2 = jnp.concatenate([h, scaled], axis=-1)
    h2 = _dotd(h2, wpost_ref[...]) + bpost_ref[...]
    h2 = g1_ref[...] * h2 / np.sqrt(1.0 + 1e-5) + beta1_ref[...]
    h2 = jnp.maximum(h2, 0.0)

    batch = batch_ref[0, 0, :]
    gids = jax.lax.broadcasted_iota(jnp.int32, (_NUM_GRAPHS, h.shape[0]), 0)
    onehot = (gids == batch[None, :]).astype(jnp.float32)
    part = jax.lax.dot(onehot, h2, precision=_HI)

    @pl.when(i == 0)
    def _():
        pooled_ref[...] = jnp.zeros_like(pooled_ref)

    pooled_ref[...] += part

    @pl.when(i == nblk - 1)
    def _():
        pooled = pooled_ref[...]
        a = jnp.maximum(_dotd(pooled, wa_ref[...]) + ba_ref[...], 0.0)
        out_ref[...] = _dotd(a, wb_ref[...]) + bb_ref[...]


def _run_c(h, hd, cnt2d, sum_q, sq_q, mx_q, mn_q, batch3d,
           Wpost, bpost, g1, beta1, Wa_p, ba_p, Wb_p, bb):
    nblk = _N // _ROW_BLK
    row_spec = pl.BlockSpec((_ROW_BLK, _F), lambda i: (i, 0))
    cnt_spec = pl.BlockSpec((_ROW_BLK, 1), lambda i: (i, 0))
    b_spec = pl.BlockSpec((1, 1, _ROW_BLK), lambda i: (i, 0, 0))
    full = lambda a: pl.BlockSpec(a.shape, lambda i: (0,) * a.ndim)
    return pl.pallas_call(
        _c_body,
        grid=(nblk,),
        in_specs=[row_spec, row_spec, cnt_spec, row_spec, row_spec,
                  row_spec, row_spec, b_spec, full(Wpost), full(bpost),
                  full(g1), full(beta1), full(Wa_p), full(ba_p),
                  full(Wb_p), full(bb)],
        out_specs=pl.BlockSpec((_NUM_GRAPHS, 1), lambda i: (0, 0)),
        out_shape=jax.ShapeDtypeStruct((_NUM_GRAPHS, 1), jnp.float32),
        scratch_shapes=[pltpu.VMEM((_NUM_GRAPHS, _F), jnp.float32)],
    )(h, hd, cnt2d, sum_q, sq_q, mx_q, mn_q, batch3d,
      Wpost, bpost, g1, beta1, Wa_p, ba_p, Wb_p, bb)


_NC = 2
_NS = 16
_NW = _NC * _NS
_RS = 160
_NPAD = _NW * 2 * _RS
_CHUNK = 2000
_NCHUNK = _E // _CHUNK
_LB = 4096
_FLUSH = 2048
_RING = 4


def _b_body(dst_hbm, src_hbm, hs_hbm, et_hbm,
            cnt_hbm, sum_hbm, sq_hbm, mx_hbm, mn_hbm,
            dst_c0, src_c0, dst_c1, src_c1, dloc_b, srcm_b, eid_b,
            hs_rows, et_rows,
            acc_sum, acc_sq, acc_mx, acc_mn, cnt_acc, sem_a, sem_b,
            sem_c, sem_d):
    wid = lax.axis_index("s") * _NC + lax.axis_index("c")
    zeros16 = jnp.zeros((16,), jnp.float32)
    ones16 = jnp.ones((16,), jnp.float32)
    neg16 = jnp.full((16,), -jnp.inf, jnp.float32)
    pos16 = jnp.full((16,), jnp.inf, jnp.float32)
    zi16 = jnp.zeros((16,), jnp.int32)
    iota16 = lax.iota(jnp.int32, 16)

    def fire(b):
        s = lax.rem(b, _RING)
        iva = srcm_b[pl.ds(b * 16, 16)]
        ivb = eid_b[pl.ds(b * 16, 16)]
        pltpu.async_copy(hs_hbm.at[iva], hs_rows.at[s], sem_a)
        pltpu.async_copy(et_hbm.at[ivb], et_rows.at[s], sem_b)

    def flush(off):
        nb = (off + 15) // 16
        srcm_b[pl.ds(off, 16)] = zi16
        eid_b[pl.ds(off, 16)] = zi16

        def prefire(b, _):
            fire(b)
            return 0

        lax.fori_loop(0, jnp.minimum(nb, _RING), prefire, 0)

        def batch_body(b, _):
            s = lax.rem(b, _RING)
            bb0 = b * 16
            ivd = srcm_b[pl.ds(bb0, 16)]
            pltpu.make_async_copy(hs_hbm.at[ivd], hs_rows.at[s], sem_a).wait()
            pltpu.make_async_copy(et_hbm.at[ivd], et_rows.at[s], sem_b).wait()
            ne = jnp.minimum(16, off - bb0)

            def edge_body(j, _):
                dloc = dloc_b[pl.ds(bb0 + j, 16)][0]
                rowb = dloc * _F
                for vi in range(_F // 16):
                    fs = pl.ds(vi * 16, 16)
                    asl = pl.ds(rowb + vi * 16, 16)
                    q = hs_rows[s, j, fs] + et_rows[s, j, fs]
                    plsc.addupdate(acc_sum.at[asl], q)
                    plsc.addupdate(acc_sq.at[asl], q * q)
                    acc_mx[asl] = jnp.maximum(acc_mx[asl], q)
                    acc_mn[asl] = jnp.minimum(acc_mn[asl], q)
                return 0

            lax.fori_loop(0, ne, edge_body, 0)

            @pl.when(b + _RING < nb)
            def _():
                fire(b + _RING)

            return 0

        lax.fori_loop(0, nb, batch_body, 0)

    def flush_reset(off):
        flush(off)
        return jnp.int32(0)

    for r_i in range(2):
        r = wid * 2 + r_i
        lo = r * _RS
        hi = lo + _RS

        def init_body(k, _):
            sl = pl.ds(k * 16, 16)
            acc_sum[sl] = zeros16
            acc_sq[sl] = zeros16
            acc_mx[sl] = neg16
            acc_mn[sl] = pos16
            return 0

        lax.fori_loop(0, _RS * _F // 16, init_body, 0)

        def cinit_body(k, _):
            cnt_acc[pl.ds(k * 16, 16)] = zeros16
            return 0

        lax.fori_loop(0, _RS // 16, cinit_body, 0)

        def fire_chunk(c, bd, bs):
            base = c * _CHUNK
            pltpu.async_copy(dst_hbm.at[pl.ds(base, _CHUNK)], bd, sem_c)
            pltpu.async_copy(src_hbm.at[pl.ds(base, _CHUNK)], bs, sem_d)

        def scan_chunk(c, bd, bs, off):
            base = c * _CHUNK
            pltpu.make_async_copy(
                dst_hbm.at[pl.ds(base, _CHUNK)], bd, sem_c).wait()
            pltpu.make_async_copy(
                src_hbm.at[pl.ds(base, _CHUNK)], bs, sem_d).wait()

            def scan_body(v, off):
                sl = pl.ds(v * 16, 16)
                d = bd[sl]
                msk = (d >= lo) & (d < hi)

                def compact(off):
                    s = bs[sl]
                    dl = d - lo
                    cs = plsc.cumsum(msk.astype(jnp.int32))
                    pos = off + cs - 1
                    plsc.store_scatter(dloc_b, [pos], dl, mask=msk)
                    plsc.store_scatter(srcm_b, [pos], s, mask=msk)
                    eid = (base + v * 16) + iota16
                    plsc.store_scatter(eid_b, [pos], eid, mask=msk)
                    plsc.addupdate_scatter(cnt_acc, [dl], ones16, mask=msk)
                    return off + cs[15]

                return lax.cond(jnp.any(msk), compact, lambda o: o, off)

            off = lax.fori_loop(0, _CHUNK // 16, scan_body, off)
            return lax.cond(off >= _FLUSH, flush_reset, lambda o: o, off)

        fire_chunk(0, dst_c0, src_c0)

        def pair_body(i, off):
            c0 = 2 * i
            fire_chunk(c0 + 1, dst_c1, src_c1)
            off = scan_chunk(c0, dst_c0, src_c0, off)

            @pl.when(c0 + 2 < _NCHUNK)
            def _():
                fire_chunk(c0 + 2, dst_c0, src_c0)

            off = scan_chunk(c0 + 1, dst_c1, src_c1, off)
            return off

        off_fin = lax.fori_loop(0, _NCHUNK // 2, pair_body, jnp.int32(0))
        lax.cond(off_fin > 0, flush_reset, lambda o: o, off_fin)

        pltpu.sync_copy(cnt_acc, cnt_hbm.at[pl.ds(lo, _RS)])
        pltpu.sync_copy(acc_sum, sum_hbm.at[pl.ds(lo * _F, _RS * _F)])
        pltpu.sync_copy(acc_sq, sq_hbm.at[pl.ds(lo * _F, _RS * _F)])
        pltpu.sync_copy(acc_mx, mx_hbm.at[pl.ds(lo * _F, _RS * _F)])
        pltpu.sync_copy(acc_mn, mn_hbm.at[pl.ds(lo * _F, _RS * _F)])


def _run_b(dst, src, hs, et):
    kern = pl.kernel(
        _b_body,
        out_type=[
            jax.ShapeDtypeStruct((_NPAD,), jnp.float32),
            jax.ShapeDtypeStruct((_NPAD * _F,), jnp.float32),
            jax.ShapeDtypeStruct((_NPAD * _F,), jnp.float32),
            jax.ShapeDtypeStruct((_NPAD * _F,), jnp.float32),
            jax.ShapeDtypeStruct((_NPAD * _F,), jnp.float32),
        ],
        mesh=plsc.VectorSubcoreMesh(
            core_axis_name="c", subcore_axis_name="s",
            num_cores=_NC, num_subcores=_NS),
        compiler_params=pltpu.CompilerParams(needs_layout_passes=False),
        scratch_types=[
            pltpu.VMEM((_CHUNK,), jnp.int32),
            pltpu.VMEM((_CHUNK,), jnp.int32),
            pltpu.VMEM((_CHUNK,), jnp.int32),
            pltpu.VMEM((_CHUNK,), jnp.int32),
            pltpu.VMEM((_LB,), jnp.int32),
            pltpu.VMEM((_LB,), jnp.int32),
            pltpu.VMEM((_LB,), jnp.int32),
            pltpu.VMEM((_RING, 16, _F), jnp.float32),
            pltpu.VMEM((_RING, 16, _F), jnp.float32),
            pltpu.VMEM((_RS * _F,), jnp.float32),
            pltpu.VMEM((_RS * _F,), jnp.float32),
            pltpu.VMEM((_RS * _F,), jnp.float32),
            pltpu.VMEM((_RS * _F,), jnp.float32),
            pltpu.VMEM((_RS,), jnp.float32),
            pltpu.SemaphoreType.DMA,
            pltpu.SemaphoreType.DMA,
            pltpu.SemaphoreType.DMA,
            pltpu.SemaphoreType.DMA,
        ],
    )
    cnt_p, sum_p, sq_p, mx_p, mn_p = kern(dst, src, hs, et)
    cnt = cnt_p[:_N]
    sum_q = sum_p.reshape(_NPAD, _F)[:_N]
    sq_q = sq_p.reshape(_NPAD, _F)[:_N]
    mx_q = mx_p.reshape(_NPAD, _F)[:_N]
    mn_q = mn_p.reshape(_NPAD, _F)[:_N]
    return cnt, sum_q, sq_q, mx_q, mn_q


def kernel(x, edge_index, edge_attr, batch, W1, b1, W2, b2, We, be, Wp, bp,
           Wpost, bpost, g1, beta1, Wa, ba, Wb, bb):
    Wpd = Wp[:_F]
    Wps = Wp[_F:2 * _F]
    Wpe = Wp[2 * _F:]
    b1r = b1.reshape(1, -1)
    b2r = b2.reshape(1, -1)
    ber = be.reshape(1, -1)
    bpr = bp.reshape(1, -1)
    bpostr = bpost.reshape(1, -1)
    g1r = g1.reshape(1, -1)
    beta1r = beta1.reshape(1, -1)
    Wa_p = jnp.pad(Wa, ((0, 0), (0, _F - Wa.shape[1])))
    ba_p = jnp.pad(ba, ((0, _F - ba.shape[0]))).reshape(1, -1)
    Wb_p = jnp.pad(Wb, ((0, _F - Wb.shape[0]), (0, 0)))
    bbr = bb.reshape(1, -1)
    batch3d = batch.reshape(_N // _ROW_BLK, 1, _ROW_BLK)

    h, hd, hs = _run_a1(x, W1, b1r, W2, b2r, Wpd, Wps)
    et = _run_a2(edge_attr, We, Wpe, ber, bpr)
    cnt, sum_q, sq_q, mx_q, mn_q = _run_b(edge_index[1], edge_index[0], hs, et)
    out = _run_c(h, hd, cnt.reshape(_N, 1), sum_q, sq_q, mx_q, mn_q,
                 batch3d, Wpost, bpostr, g1r, beta1r, Wa_p, ba_p, Wb_p, bbr)
    return out


# speedup vs baseline: 1.1480x; 1.0091x over previous
import functools

import jax
import jax.numpy as jnp
import numpy as np
from jax import lax
from jax.experimental import pallas as pl
from jax.experimental.pallas import tpu as pltpu
from jax.experimental.pallas import tpu_sc as plsc

_N = 10000
_E = 320000
_F = 128
_EDGE_DIM = 16
_NUM_GRAPHS = 64
_AVG_LOG_DEG = float(np.log(33.0))
_HI = jax.lax.Precision.HIGHEST


def _dotd(a, b):
    return jax.lax.dot_general(
        a.astype(jnp.bfloat16), b.astype(jnp.bfloat16),
        (((1,), (0,)), ((), ())), preferred_element_type=jnp.float32)

_ROW_BLK = 1000
_EDGE_BLK = 4000


def _a1_body(x_ref, w1_ref, b1_ref, w2_ref, b2_ref, wpd_ref, wps_ref,
             h_ref, hd_ref, hs_ref):
    x = x_ref[...]
    hmid = jnp.maximum(_dotd(x, w1_ref[...]) + b1_ref[...], 0.0)
    h = _dotd(hmid, w2_ref[...]) + b2_ref[...]
    h_ref[...] = h
    hd_ref[...] = _dotd(h, wpd_ref[...])
    hs_ref[...] = _dotd(h, wps_ref[...])


def _run_a1(x, W1, b1, W2, b2, Wpd, Wps):
    nblk = _N // _ROW_BLK
    row_spec = pl.BlockSpec((_ROW_BLK, _F), lambda i: (i, 0))
    full = lambda a: pl.BlockSpec(a.shape, lambda i: (0,) * a.ndim)
    out_sd = jax.ShapeDtypeStruct((_N, _F), jnp.float32)
    return pl.pallas_call(
        _a1_body,
        grid=(nblk,),
        in_specs=[row_spec, full(W1), full(b1), full(W2), full(b2),
                  full(Wpd), full(Wps)],
        out_specs=[row_spec, row_spec, row_spec],
        out_shape=[out_sd, out_sd, out_sd],
    )(x, W1, b1, W2, b2, Wpd, Wps)


def _a2_body(ea_ref, we_ref, wpe_ref, be_ref, bp_ref, et_ref):
    e = _dotd(ea_ref[...], we_ref[...]) + be_ref[...]
    et_ref[...] = _dotd(e, wpe_ref[...]) + bp_ref[...]


def _run_a2(edge_attr, We, Wpe, be, bp):
    nblk = _E // _EDGE_BLK
    full = lambda a: pl.BlockSpec(a.shape, lambda i: (0,) * a.ndim)
    return pl.pallas_call(
        _a2_body,
        grid=(nblk,),
        in_specs=[pl.BlockSpec((_EDGE_BLK, _EDGE_DIM), lambda i: (i, 0)),
                  full(We), full(Wpe), full(be), full(bp)],
        out_specs=pl.BlockSpec((_EDGE_BLK, _F), lambda i: (i, 0)),
        out_shape=jax.ShapeDtypeStruct((_E, _F), jnp.float32),
    )(edge_attr, We, Wpe, be, bp)


def _c_body(h_ref, hd_ref, cnt_ref, sum_ref, sq_ref, mx_ref, mn_ref,
            batch_ref, wpost_ref, bpost_ref, g1_ref, beta1_ref,
            wa_ref, ba_ref, wb_ref, bb_ref, out_ref, pooled_ref):
    i = pl.program_id(0)
    nblk = pl.num_programs(0)

    h = h_ref[...]
    hd = hd_ref[...]
    cnt = cnt_ref[...]
    sum_q = sum_ref[...]
    sq_q = sq_ref[...]
    cnt_c = jnp.maximum(cnt, 1.0)
    sum_m = sum_q + cnt * hd
    mean = sum_m / cnt_c
    mean_sq = (sq_q + 2.0 * hd * sum_q + cnt * hd * hd) / cnt_c
    std = jnp.sqrt(jnp.maximum(mean_sq - mean * mean, 0.0) + 1e-5)
    has = cnt > 0.0
    mx = jnp.where(has, hd + mx_ref[...], 0.0)
    mn = jnp.where(has, hd + mn_ref[...], 0.0)
    aggs = jnp.concatenate([mean, mn, mx, std], axis=-1)
    logd = jnp.log(cnt + 1.0)
    amp = logd / _AVG_LOG_DEG
    safe_logd = jnp.where(logd > 0.0, logd, 1.0)
    att = jnp.where(logd > 0.0, _AVG_LOG_DEG / safe_logd, 1.0)
    scaled = jnp.concatenate([aggs, aggs * amp, aggs * att], axis=-1)
    h2 = jnp.concatenate([h, scaled], axis=-1)
    h2 = _dotd(h2, wpost_ref[...]) + bpost_ref[...]
    h2 = g1_ref[...] * h2 / np.sqrt(1.0 + 1e-5) + beta1_ref[...]
    h2 = jnp.maximum(h2, 0.0)

    batch = batch_ref[0, 0, :]
    gids = jax.lax.broadcasted_iota(jnp.int32, (_NUM_GRAPHS, h.shape[0]), 0)
    onehot = (gids == batch[None, :]).astype(jnp.float32)
    part = jax.lax.dot(onehot, h2, precision=_HI)

    @pl.when(i == 0)
    def _():
        pooled_ref[...] = jnp.zeros_like(pooled_ref)

    pooled_ref[...] += part

    @pl.when(i == nblk - 1)
    def _():
        pooled = pooled_ref[...]
        a = jnp.maximum(_dotd(pooled, wa_ref[...]) + ba_ref[...], 0.0)
        out_ref[...] = _dotd(a, wb_ref[...]) + bb_ref[...]


def _run_c(h, hd, cnt2d, sum_q, sq_q, mx_q, mn_q, batch3d,
           Wpost, bpost, g1, beta1, Wa_p, ba_p, Wb_p, bb):
    nblk = _N // _ROW_BLK
    row_spec = pl.BlockSpec((_ROW_BLK, _F), lambda i: (i, 0))
    cnt_spec = pl.BlockSpec((_ROW_BLK, 1), lambda i: (i, 0))
    b_spec = pl.BlockSpec((1, 1, _ROW_BLK), lambda i: (i, 0, 0))
    full = lambda a: pl.BlockSpec(a.shape, lambda i: (0,) * a.ndim)
    return pl.pallas_call(
        _c_body,
        grid=(nblk,),
        in_specs=[row_spec, row_spec, cnt_spec, row_spec, row_spec,
                  row_spec, row_spec, b_spec, full(Wpost), full(bpost),
                  full(g1), full(beta1), full(Wa_p), full(ba_p),
                  full(Wb_p), full(bb)],
        out_specs=pl.BlockSpec((_NUM_GRAPHS, 1), lambda i: (0, 0)),
        out_shape=jax.ShapeDtypeStruct((_NUM_GRAPHS, 1), jnp.float32),
        scratch_shapes=[pltpu.VMEM((_NUM_GRAPHS, _F), jnp.float32)],
    )(h, hd, cnt2d, sum_q, sq_q, mx_q, mn_q, batch3d,
      Wpost, bpost, g1, beta1, Wa_p, ba_p, Wb_p, bb)


_NC = 2
_NS = 16
_NW = _NC * _NS
_RS = 160
_NPAD = _NW * 2 * _RS
_CHUNK = 2000
_NCHUNK = _E // _CHUNK
_LB = 4096
_FLUSH = 2048
_RING = 4


def _b_body(dst_hbm, src_hbm, hs_hbm, et_hbm,
            cnt_hbm, sum_hbm, sq_hbm, mx_hbm, mn_hbm,
            dst_c0, src_c0, dst_c1, src_c1, dloc_b, srcm_b, eid_b,
            hs_rows, et_rows,
            acc_sum, acc_sq, acc_mx, acc_mn, cnt_acc, sem_a, sem_b,
            sem_c, sem_d):
    wid = lax.axis_index("s") * _NC + lax.axis_index("c")
    zeros16 = jnp.zeros((16,), jnp.float32)
    ones16 = jnp.ones((16,), jnp.float32)
    neg16 = jnp.full((16,), -jnp.inf, jnp.float32)
    pos16 = jnp.full((16,), jnp.inf, jnp.float32)
    zi16 = jnp.zeros((16,), jnp.int32)
    iota16 = lax.iota(jnp.int32, 16)

    def fire(b):
        s = lax.rem(b, _RING)
        iva = srcm_b[pl.ds(b * 16, 16)]
        ivb = eid_b[pl.ds(b * 16, 16)]
        pltpu.async_copy(hs_hbm.at[iva], hs_rows.at[s], sem_a)
        pltpu.async_copy(et_hbm.at[ivb], et_rows.at[s], sem_b)

    def flush(off):
        nb = (off + 15) // 16
        srcm_b[pl.ds(off, 16)] = zi16
        eid_b[pl.ds(off, 16)] = zi16

        def prefire(b, _):
            fire(b)
            return 0

        lax.fori_loop(0, jnp.minimum(nb, _RING), prefire, 0)

        def batch_body(b, _):
            s = lax.rem(b, _RING)
            bb0 = b * 16
            ivd = srcm_b[pl.ds(bb0, 16)]
            pltpu.make_async_copy(hs_hbm.at[ivd], hs_rows.at[s], sem_a).wait()
            pltpu.make_async_copy(et_hbm.at[ivd], et_rows.at[s], sem_b).wait()
            ne = jnp.minimum(16, off - bb0)

            def edge_body(j, _):
                dloc = dloc_b[pl.ds(bb0 + j, 16)][0]
                rowb = dloc * _F
                for vi in range(_F // 16):
                    fs = pl.ds(vi * 16, 16)
                    asl = pl.ds(rowb + vi * 16, 16)
                    q = hs_rows[s, j, fs] + et_rows[s, j, fs]
                    plsc.addupdate(acc_sum.at[asl], q)
                    plsc.addupdate(acc_sq.at[asl], q * q)
                    acc_mx[asl] = jnp.maximum(acc_mx[asl], q)
                    acc_mn[asl] = jnp.minimum(acc_mn[asl], q)
                return 0

            lax.fori_loop(0, ne, edge_body, 0)

            @pl.when(b + _RING < nb)
            def _():
                fire(b + _RING)

            return 0

        lax.fori_loop(0, nb, batch_body, 0)

    def flush_reset(off):
        flush(off)
        return jnp.int32(0)

    for r_i in range(2):
        r = wid * 2 + r_i
        lo = r * _RS
        hi = lo + _RS

        def init_body(k, _):
            sl = pl.ds(k * 16, 16)
            acc_sum[sl] = zeros16
            acc_sq[sl] = zeros16
            acc_mx[sl] = neg16
            acc_mn[sl] = pos16
            return 0

        lax.fori_loop(0, _RS * _F // 16, init_body, 0)

        def cinit_body(k, _):
            cnt_acc[pl.ds(k * 16, 16)] = zeros16
            return 0

        lax.fori_loop(0, _RS // 16, cinit_body, 0)

        def fire_chunk(c, bd, bs):
            base = c * _CHUNK
            pltpu.async_copy(dst_hbm.at[pl.ds(base, _CHUNK)], bd, sem_c)
            pltpu.async_copy(src_hbm.at[pl.ds(base, _CHUNK)], bs, sem_d)

        def scan_chunk(c, bd, bs, off):
            base = c * _CHUNK
            pltpu.make_async_copy(
                dst_hbm.at[pl.ds(base, _CHUNK)], bd, sem_c).wait()
            pltpu.make_async_copy(
                src_hbm.at[pl.ds(base, _CHUNK)], bs, sem_d).wait()

            def scan_body(v, off):
                sl = pl.ds(v * 16, 16)
                d = bd[sl]
                msk = (d >= lo) & (d < hi)

                def compact(off):
                    s = bs[sl]
                    dl = d - lo
                    cs = plsc.cumsum(msk.astype(jnp.int32))
                    pos = off + cs - 1
                    plsc.store_scatter(dloc_b, [pos], dl, mask=msk)
                    plsc.store_scatter(srcm_b, [pos], s, mask=msk)
                    eid = (base + v * 16) + iota16
                    plsc.store_scatter(eid_b, [pos], eid, mask=msk)
                    plsc.addupdate_scatter(cnt_acc, [dl], ones16, mask=msk)
                    return off + cs[15]

                return lax.cond(jnp.any(msk), compact, lambda o: o, off)

            off = lax.fori_loop(0, _CHUNK // 16, scan_body, off, unroll=4)
            return lax.cond(off >= _FLUSH, flush_reset, lambda o: o, off)

        fire_chunk(0, dst_c0, src_c0)

        def pair_body(i, off):
            c0 = 2 * i
            fire_chunk(c0 + 1, dst_c1, src_c1)
            off = scan_chunk(c0, dst_c0, src_c0, off)

            @pl.when(c0 + 2 < _NCHUNK)
            def _():
                fire_chunk(c0 + 2, dst_c0, src_c0)

            off = scan_chunk(c0 + 1, dst_c1, src_c1, off)
            return off

        off_fin = lax.fori_loop(0, _NCHUNK // 2, pair_body, jnp.int32(0))
        lax.cond(off_fin > 0, flush_reset, lambda o: o, off_fin)

        pltpu.sync_copy(cnt_acc, cnt_hbm.at[pl.ds(lo, _RS)])
        pltpu.sync_copy(acc_sum, sum_hbm.at[pl.ds(lo * _F, _RS * _F)])
        pltpu.sync_copy(acc_sq, sq_hbm.at[pl.ds(lo * _F, _RS * _F)])
        pltpu.sync_copy(acc_mx, mx_hbm.at[pl.ds(lo * _F, _RS * _F)])
        pltpu.sync_copy(acc_mn, mn_hbm.at[pl.ds(lo * _F, _RS * _F)])


def _run_b(dst, src, hs, et):
    kern = pl.kernel(
        _b_body,
        out_type=[
            jax.ShapeDtypeStruct((_NPAD,), jnp.float32),
            jax.ShapeDtypeStruct((_NPAD * _F,), jnp.float32),
            jax.ShapeDtypeStruct((_NPAD * _F,), jnp.float32),
            jax.ShapeDtypeStruct((_NPAD * _F,), jnp.float32),
            jax.ShapeDtypeStruct((_NPAD * _F,), jnp.float32),
        ],
        mesh=plsc.VectorSubcoreMesh(
            core_axis_name="c", subcore_axis_name="s",
            num_cores=_NC, num_subcores=_NS),
        compiler_params=pltpu.CompilerParams(needs_layout_passes=False),
        scratch_types=[
            pltpu.VMEM((_CHUNK,), jnp.int32),
            pltpu.VMEM((_CHUNK,), jnp.int32),
            pltpu.VMEM((_CHUNK,), jnp.int32),
            pltpu.VMEM((_CHUNK,), jnp.int32),
            pltpu.VMEM((_LB,), jnp.int32),
            pltpu.VMEM((_LB,), jnp.int32),
            pltpu.VMEM((_LB,), jnp.int32),
            pltpu.VMEM((_RING, 16, _F), jnp.float32),
            pltpu.VMEM((_RING, 16, _F), jnp.float32),
            pltpu.VMEM((_RS * _F,), jnp.float32),
            pltpu.VMEM((_RS * _F,), jnp.float32),
            pltpu.VMEM((_RS * _F,), jnp.float32),
            pltpu.VMEM((_RS * _F,), jnp.float32),
            pltpu.VMEM((_RS,), jnp.float32),
            pltpu.SemaphoreType.DMA,
            pltpu.SemaphoreType.DMA,
            pltpu.SemaphoreType.DMA,
            pltpu.SemaphoreType.DMA,
        ],
    )
    cnt_p, sum_p, sq_p, mx_p, mn_p = kern(dst, src, hs, et)
    cnt = cnt_p[:_N]
    sum_q = sum_p.reshape(_NPAD, _F)[:_N]
    sq_q = sq_p.reshape(_NPAD, _F)[:_N]
    mx_q = mx_p.reshape(_NPAD, _F)[:_N]
    mn_q = mn_p.reshape(_NPAD, _F)[:_N]
    return cnt, sum_q, sq_q, mx_q, mn_q


def kernel(x, edge_index, edge_attr, batch, W1, b1, W2, b2, We, be, Wp, bp,
           Wpost, bpost, g1, beta1, Wa, ba, Wb, bb):
    Wpd = Wp[:_F]
    Wps = Wp[_F:2 * _F]
    Wpe = Wp[2 * _F:]
    b1r = b1.reshape(1, -1)
    b2r = b2.reshape(1, -1)
    ber = be.reshape(1, -1)
    bpr = bp.reshape(1, -1)
    bpostr = bpost.reshape(1, -1)
    g1r = g1.reshape(1, -1)
    beta1r = beta1.reshape(1, -1)
    Wa_p = jnp.pad(Wa, ((0, 0), (0, _F - Wa.shape[1])))
    ba_p = jnp.pad(ba, ((0, _F - ba.shape[0]))).reshape(1, -1)
    Wb_p = jnp.pad(Wb, ((0, _F - Wb.shape[0]), (0, 0)))
    bbr = bb.reshape(1, -1)
    batch3d = batch.reshape(_N // _ROW_BLK, 1, _ROW_BLK)

    h, hd, hs = _run_a1(x, W1, b1r, W2, b2r, Wpd, Wps)
    et = _run_a2(edge_attr, We, Wpe, ber, bpr)
    cnt, sum_q, sq_q, mx_q, mn_q = _run_b(edge_index[1], edge_index[0], hs, et)
    out = _run_c(h, hd, cnt.reshape(_N, 1), sum_q, sq_q, mx_q, mn_q,
                 batch3d, Wpost, bpostr, g1r, beta1r, Wa_p, ba_p, Wb_p, bbr)
    return out
